# Initial kernel scaffold; baseline (speedup 1.0000x reference)
import jax
import jax.numpy as jnp
from jax.experimental import pallas as pl


def kernel(prp_boxes, prp_labels, tgt_boxes, tgt_labels, tgt_rel_matrix):
    raise NotImplementedError("write your pallas kernel here")


# speedup vs baseline: 4.4957x; 4.4957x over previous
import functools

import jax
import jax.numpy as jnp
from jax import lax
from jax.experimental import pallas as pl
from jax.experimental.pallas import tpu as pltpu

P = 512
T = 64
K_FG = 256
K_BG = 768
FG_THRES = 0.5

_HIGH = lax.Precision.HIGHEST
_f32 = jnp.float32
_i32 = jnp.int32


def _mm(a, b):
    return lax.dot_general(a, b, (((1,), (0,)), ((), ())),
                           precision=_HIGH, preferred_element_type=_f32)


def _iota(n0, n1, d):
    return lax.broadcasted_iota(_i32, (n0, n1), d)


def _eye(n):
    return (_iota(n, n, 0) == _iota(n, n, 1)).astype(_f32)


def _row_from_col(col, n):
    return lax.dot_general(col, _eye(n), (((0,), (0,)), ((), ())),
                           precision=_HIGH, preferred_element_type=_f32)


def _col_from_row(row, eye_n):
    return lax.dot_general(eye_n, row, (((1,), (1,)), ((), ())),
                           precision=_HIGH, preferred_element_type=_f32)


def _select_topk(K, k):
    kf = _f32(k)
    def bs_body(_, carry):
        lo, hi = carry
        mid = lo + lax.shift_right_logical(hi - lo + 1, 1)
        cnt = jnp.sum((K >= mid).astype(_f32))
        pred = cnt >= kf
        return (jnp.where(pred, mid, lo), jnp.where(pred, hi, mid - 1))

    lo0 = jnp.int32(-1)
    hi0 = jnp.int32(1 << 30)
    tau, _ = lax.fori_loop(0, 31, bs_body, (lo0, hi0))

    gt = (K > tau).astype(_f32)
    eq = (K == tau).astype(_f32)
    n_gt = jnp.sum(gt)
    need = kf - n_gt

    U = (_iota(P, P, 0) < _iota(P, P, 1)).astype(_f32)
    Lst = (_iota(P, P, 0) > _iota(P, P, 1)).astype(_f32)

    w_eq = _mm(eq, U)
    rowbase_eq = _mm(Lst, jnp.sum(eq, axis=1, keepdims=True))
    eq_prefix = rowbase_eq + w_eq
    sel = gt + eq * (eq_prefix < need).astype(_f32)

    w_sel = _mm(sel, U)
    rowcnt = jnp.sum(sel, axis=1, keepdims=True)
    rowbase = _mm(Lst, rowcnt)
    rowend_row = _row_from_col(rowbase + rowcnt, P)

    s_col = _iota(k, 1, 0).astype(_f32)
    r_col = jnp.sum((rowend_row <= s_col).astype(_f32), axis=1, keepdims=True)
    lane_k = _iota(k, P, 1).astype(_f32)
    onehot_r = (lane_k == r_col).astype(_f32)

    rbase_s = _mm(onehot_r, rowbase)
    gsel = _mm(onehot_r, sel)
    gw = _mm(onehot_r, w_sel)
    ind = gsel * (gw == (s_col - rbase_s)).astype(_f32)
    c_col = jnp.sum(ind * lane_k, axis=1, keepdims=True)

    khi = jnp.right_shift(K, 12).astype(_f32)
    klo = (K & 0xFFF).astype(_f32)
    khi_c = jnp.sum(_mm(onehot_r, khi) * ind, axis=1, keepdims=True)
    klo_c = jnp.sum(_mm(onehot_r, klo) * ind, axis=1, keepdims=True)
    khi_r = _row_from_col(khi_c, k)
    klo_r = _row_from_col(klo_c, k)

    j_col = _iota(k, 1, 0)
    i_row = _iota(1, k, 1)
    cgt = (khi_c > khi_r) | ((khi_c == khi_r) & (klo_c > klo_r))
    ceq = (khi_c == khi_r) & (klo_c == klo_r)
    cmpm = cgt | (ceq & (j_col < i_row))
    rank_row = jnp.sum(cmpm.astype(_f32), axis=0, keepdims=True)
    pmat = (rank_row == _iota(k, 1, 0).astype(_f32)).astype(_f32)

    eye_k = _eye(k)

    def order(col):
        row = lax.dot_general(col, eye_k, (((0,), (0,)), ((), ())),
                              precision=_HIGH, preferred_element_type=_f32)
        return jnp.sum(pmat * row, axis=1, keepdims=True)

    def order_row(row):
        return jnp.sum(pmat * row, axis=1, keepdims=True)

    def gather(x):
        return jnp.sum(_mm(onehot_r, x) * ind, axis=1, keepdims=True)

    return dict(order=order, order_row=order_row, gather=gather,
                r_col=r_col, c_col=c_col, pmat=pmat)


def _body(tb_ref, tbT_ref, pb_ref, pbT_ref, tl_col_ref, tl_row_ref,
          plab_row_ref, plab_col_ref, relf_ref, relfT_ref, nbits_ref,
          pairs_ref, labels_ref, binary_ref, qual_ref,
          ious_s, relposT_s, inner_s, ioupair_s):
    tb = tb_ref[...]
    tbT = tbT_ref[...]
    pb = pb_ref[...]
    pbT = pbT_ref[...]
    tl_col = tl_col_ref[...]
    tl_row = tl_row_ref[...]
    plab_row = plab_row_ref[...]
    plab_col = plab_col_ref[...]
    relf = relf_ref[...]
    relfT = relfT_ref[...]
    nbits = nbits_ref[...]

    def iou(ac, bc):
        a0, a1, a2, a3 = ac
        b0, b1, b2, b3 = bc
        area_a = (a2 - a0) * (a3 - a1)
        area_b = (b2 - b0) * (b3 - b1)
        w = jnp.maximum(jnp.minimum(a2, b2) - jnp.maximum(a0, b0), 0.0)
        h = jnp.maximum(jnp.minimum(a3, b3) - jnp.maximum(a1, b1), 0.0)
        inter = w * h
        union = area_a + area_b - inter
        return inter / jnp.maximum(union, 1e-8)

    t_cols = [tb[:, i:i + 1] for i in range(4)]
    p_rows = [pbT[i:i + 1, :] for i in range(4)]
    p_cols = [pb[:, i:i + 1] for i in range(4)]
    t_rows = [tbT[i:i + 1, :] for i in range(4)]
    ious = iou(t_cols, p_rows)
    iousT = iou(p_cols, t_rows)

    m = ((tl_col == plab_row) & (ious > FG_THRES)).astype(_f32)
    mT = ((plab_col == tl_row) & (iousT > FG_THRES)).astype(_f32)

    rel_pos = (relf > 0).astype(_f32)
    rel_posT = (relfT > 0).astype(_f32)

    F = _mm(_mm(mT, rel_pos), m)
    G = _mm(_mm(mT, rel_posT), m)
    binary_ref[...] = ((F + G) > 0).astype(_i32)
    eyeP = _eye(P)
    fg_mask = ((F > 0).astype(_f32)) * (1.0 - eyeP)

    ious_s[...] = ious
    relposT_s[...] = rel_posT
    inner_s[...] = jnp.zeros((T, P), _f32)
    eye_t = _eye(T)

    def in_body(t, _):
        rowm = relposT_s[pl.ds(t, 1), :]
        colm = _col_from_row(rowm, eye_t)
        rowi = ious_s[pl.ds(t, 1), :]
        inner_s[...] = jnp.maximum(inner_s[...], colm * rowi)
        return 0

    lax.fori_loop(0, T, in_body, 0)

    ioupair_s[...] = jnp.zeros((P, P), _f32)
    eyeP_v = eyeP

    def ip_body(h, _):
        rowp = ious_s[pl.ds(h, 1), :]
        colp = _col_from_row(rowp, eyeP_v)
        rowq = inner_s[pl.ds(h, 1), :]
        ioupair_s[...] = jnp.maximum(ioupair_s[...], colp * rowq)
        return 0

    lax.fori_loop(0, T, ip_body, 0)
    iou_pair = ioupair_s[...]

    fg_scores = fg_mask * iou_pair
    k_fg = lax.bitcast_convert_type(fg_scores, _i32)

    s_fg = _select_topk(k_fg, K_FG)
    fg_head = s_fg["order"](s_fg["r_col"])
    fg_tail = s_fg["order"](s_fg["c_col"])
    fg_val = s_fg["order"](s_fg["gather"](fg_scores))

    r_row = _row_from_col(s_fg["r_col"], K_FG)
    c_row = _row_from_col(s_fg["c_col"], K_FG)
    onehot_rT = (_iota(P, K_FG, 0).astype(_f32) == r_row).astype(_f32)
    onehot_cT = (_iota(P, K_FG, 0).astype(_f32) == c_row).astype(_f32)
    a_headT = _mm(m, onehot_rT)
    a_tailT = _mm(m, onehot_cT)
    t1 = jnp.max(relf[:, :, None] * a_headT[:, None, :], axis=0)
    lab_row = jnp.max(t1 * a_tailT, axis=0, keepdims=True)
    lab_ord = s_fg["order_row"](lab_row)

    fg_valid = (fg_val > 0).astype(_f32)
    fg_lab = lab_ord * fg_valid
    quality = fg_val * fg_valid

    valid_p = (plab_col != 0) & (plab_row != 0)
    bgmask = valid_p & (eyeP == 0.0) & (fg_mask == 0.0)
    k_bg = jnp.where(bgmask, nbits, -1)
    s_bg = _select_topk(k_bg, K_BG)
    bg_head = s_bg["order"](s_bg["r_col"])
    bg_tail = s_bg["order"](s_bg["c_col"])

    pairs_ref[0:K_FG, 0:1] = fg_head.astype(_i32)
    pairs_ref[0:K_FG, 1:2] = fg_tail.astype(_i32)
    pairs_ref[K_FG:, 0:1] = bg_head.astype(_i32)
    pairs_ref[K_FG:, 1:2] = bg_tail.astype(_i32)
    labels_ref[0:K_FG, :] = fg_lab.astype(_i32)
    labels_ref[K_FG:, :] = jnp.zeros((K_BG, 1), _i32)
    qual_ref[...] = quality


@jax.jit
def kernel(prp_boxes, prp_labels, tgt_boxes, tgt_labels, tgt_rel_matrix):
    noise = jax.random.uniform(jax.random.key(42), (P * P,))
    nbits = lax.bitcast_convert_type(noise, _i32).reshape(P, P)
    relf = tgt_rel_matrix.astype(_f32)
    args = (
        tgt_boxes.astype(_f32),
        tgt_boxes.astype(_f32).T,
        prp_boxes.astype(_f32),
        prp_boxes.astype(_f32).T,
        tgt_labels.astype(_i32).reshape(T, 1),
        tgt_labels.astype(_i32).reshape(1, T),
        prp_labels.astype(_i32).reshape(1, P),
        prp_labels.astype(_i32).reshape(P, 1),
        relf,
        relf.T,
        nbits,
    )
    out_shape = (
        jax.ShapeDtypeStruct((K_FG + K_BG, 2), _i32),
        jax.ShapeDtypeStruct((K_FG + K_BG, 1), _i32),
        jax.ShapeDtypeStruct((P, P), _i32),
        jax.ShapeDtypeStruct((K_FG, 1), _f32),
    )
    scratch = [
        pltpu.VMEM((T, P), _f32),
        pltpu.VMEM((T, T), _f32),
        pltpu.VMEM((T, P), _f32),
        pltpu.VMEM((P, P), _f32),
    ]
    pairs, labels, binary, qual = pl.pallas_call(
        _body, out_shape=out_shape, scratch_shapes=scratch)(*args)
    return pairs, labels.reshape(K_FG + K_BG), binary, qual.reshape(K_FG)


# speedup vs baseline: 6.2377x; 1.3875x over previous
import jax
import jax.numpy as jnp
from jax import lax
from jax.experimental import pallas as pl
from jax.experimental.pallas import tpu as pltpu

P = 512
T = 64
K_FG = 256
K_BG = 768
FG_THRES = 0.5

_f32 = jnp.float32
_i32 = jnp.int32


def _mm(a, b):
    return lax.dot_general(a, b, (((1,), (0,)), ((), ())),
                           precision=lax.Precision.HIGHEST,
                           preferred_element_type=_f32)


def _mmd(a, b):
    return lax.dot_general(a, b, (((1,), (0,)), ((), ())),
                           precision=lax.Precision.DEFAULT,
                           preferred_element_type=_f32)


def _iota(n0, n1, d):
    return lax.broadcasted_iota(_i32, (n0, n1), d)


def _eye(n):
    return (_iota(n, n, 0) == _iota(n, n, 1)).astype(_f32)


def _row_from_col(col, n):
    return lax.dot_general(col, _eye(n), (((0,), (0,)), ((), ())),
                           precision=lax.Precision.HIGHEST,
                           preferred_element_type=_f32)


def _split8(x):
    hi = jnp.floor(x * (1.0 / 256.0))
    return hi, x - 256.0 * hi


def _select_topk(K, k):
    kf = _f32(k)

    def bs_body(_, carry):
        lo, hi = carry
        mid = lo + lax.shift_right_logical(hi - lo + 1, 1)
        cnt = jnp.sum((K >= mid).astype(_f32))
        pred = cnt >= kf
        return (jnp.where(pred, mid, lo), jnp.where(pred, hi, mid - 1))

    tau, _ = lax.fori_loop(0, 31, bs_body, (jnp.int32(-1), jnp.int32(1 << 30)))

    gt = (K > tau).astype(_f32)
    eq = (K == tau).astype(_f32)
    need = kf - jnp.sum(gt)

    U = (_iota(P, P, 0) < _iota(P, P, 1)).astype(_f32)
    Lst = (_iota(P, P, 0) > _iota(P, P, 1)).astype(_f32)

    w_eq = _mmd(eq, U)
    rowbase_eq = _mm(Lst, jnp.sum(eq, axis=1, keepdims=True))
    sel = gt + eq * ((rowbase_eq + w_eq) < need).astype(_f32)

    w_sel = _mmd(sel, U)
    w_hi, w_lo = _split8(w_sel)
    rowcnt = jnp.sum(sel, axis=1, keepdims=True)
    rowbase = _mm(Lst, rowcnt)
    rowend_row = _row_from_col(rowbase + rowcnt, P)

    s_col = _iota(k, 1, 0).astype(_f32)
    r_col = jnp.sum((rowend_row <= s_col).astype(_f32), axis=1, keepdims=True)
    lane_k = _iota(k, P, 1).astype(_f32)
    onehot_r = (lane_k == r_col).astype(_f32)

    rbase_s = _mm(onehot_r, rowbase)
    t_hi, t_lo = _split8(s_col - rbase_s)
    gsel = _mmd(onehot_r, sel)
    gw_hi = _mmd(onehot_r, w_hi)
    gw_lo = _mmd(onehot_r, w_lo)
    ind = gsel * ((gw_hi == t_hi) & (gw_lo == t_lo)).astype(_f32)
    c_col = jnp.sum(ind * lane_k, axis=1, keepdims=True)

    ch = (jnp.right_shift(K, 22).astype(_f32),
          (jnp.right_shift(K, 14) & 0xFF).astype(_f32),
          (jnp.right_shift(K, 6) & 0xFF).astype(_f32),
          (K & 0x3F).astype(_f32))
    g = tuple(jnp.sum(_mmd(onehot_r, c) * ind, axis=1, keepdims=True)
              for c in ch)
    r = tuple(_row_from_col(gc, k) for gc in g)

    def lexgt(i):
        t = g[3] > r[3]
        for j in (2, 1, 0):
            t = (g[j] > r[j]) | ((g[j] == r[j]) & t)
        return t

    cgt = lexgt(0)
    ceq = ((g[0] == r[0]) & (g[1] == r[1]) &
           (g[2] == r[2]) & (g[3] == r[3]))
    cmpm = cgt | (ceq & (_iota(k, 1, 0) < _iota(1, k, 1)))
    rank_row = jnp.sum(cmpm.astype(_f32), axis=0, keepdims=True)
    pmat = (rank_row == _iota(k, 1, 0).astype(_f32)).astype(_f32)

    eye_k = _eye(k)

    def order(col):
        row = lax.dot_general(col, eye_k, (((0,), (0,)), ((), ())),
                              precision=lax.Precision.HIGHEST,
                              preferred_element_type=_f32)
        return jnp.sum(pmat * row, axis=1, keepdims=True)

    def order_row(row):
        return jnp.sum(pmat * row, axis=1, keepdims=True)

    return dict(order=order, order_row=order_row, chunks=g,
                r_col=r_col, c_col=c_col)


def _body(tb_ref, tbT_ref, pb_ref, pbT_ref, tl_col_ref, tl_row_ref,
          plab_row_ref, plab_col_ref, relf_ref, relfT_ref, nbits_ref,
          pairs_ref, labels_ref, binary_ref, qual_ref,
          ious_s, inner_s, ioupair_s):
    tb = tb_ref[...]
    tbT = tbT_ref[...]
    pb = pb_ref[...]
    pbT = pbT_ref[...]
    tl_col = tl_col_ref[...]
    tl_row = tl_row_ref[...]
    plab_row = plab_row_ref[...]
    plab_col = plab_col_ref[...]
    relf = relf_ref[...]
    relfT = relfT_ref[...]
    nbits = nbits_ref[...]

    def iou(ac, bc):
        a0, a1, a2, a3 = ac
        b0, b1, b2, b3 = bc
        area_a = (a2 - a0) * (a3 - a1)
        area_b = (b2 - b0) * (b3 - b1)
        w = jnp.maximum(jnp.minimum(a2, b2) - jnp.maximum(a0, b0), 0.0)
        h = jnp.maximum(jnp.minimum(a3, b3) - jnp.maximum(a1, b1), 0.0)
        inter = w * h
        union = area_a + area_b - inter
        return inter / jnp.maximum(union, 1e-8)

    ious = iou([tb[:, i:i + 1] for i in range(4)],
               [pbT[i:i + 1, :] for i in range(4)])
    iousT = iou([pb[:, i:i + 1] for i in range(4)],
                [tbT[i:i + 1, :] for i in range(4)])

    m = ((tl_col == plab_row) & (ious > FG_THRES)).astype(_f32)
    mT = ((plab_col == tl_row) & (iousT > FG_THRES)).astype(_f32)

    rel_pos = (relf > 0).astype(_f32)
    rel_posT = (relfT > 0).astype(_f32)

    F = _mmd(_mmd(mT, rel_pos), m)
    G = _mmd(_mmd(mT, rel_posT), m)
    binary_ref[...] = ((F + G) > 0).astype(_i32)
    eyeP = _eye(P)
    fg_mask = ((F > 0).astype(_f32)) * (1.0 - eyeP)

    inner_s[...] = jnp.max(rel_posT[:, :, None] * ious[:, None, :], axis=0)
    ious_s[...] = ious

    ioupair_s[...] = jnp.zeros((P, P), _f32)

    def ip_body(i, _):
        blk = ious_s[pl.ds(i * 8, 8), :]
        ib = inner_s[pl.ds(i * 8, 8), :]
        mx = jnp.max(blk[:, :, None] * ib[:, None, :], axis=0)
        ioupair_s[...] = jnp.maximum(ioupair_s[...], mx)
        return 0

    lax.fori_loop(0, T // 8, ip_body, 0)
    iou_pair = ioupair_s[...]

    fg_scores = fg_mask * iou_pair
    k_fg = lax.bitcast_convert_type(fg_scores, _i32)

    s_fg = _select_topk(k_fg, K_FG)
    fg_head = s_fg["order"](s_fg["r_col"])
    fg_tail = s_fg["order"](s_fg["c_col"])
    o3, o2, o1, o0 = (s_fg["order"](gc) for gc in s_fg["chunks"])
    k_ord = (lax.shift_left(o3.astype(_i32), 22) |
             lax.shift_left(o2.astype(_i32), 14) |
             lax.shift_left(o1.astype(_i32), 6) | o0.astype(_i32))
    fg_val = lax.bitcast_convert_type(k_ord, _f32)

    r_row = _row_from_col(s_fg["r_col"], K_FG)
    c_row = _row_from_col(s_fg["c_col"], K_FG)
    onehot_rT = (_iota(P, K_FG, 0).astype(_f32) == r_row).astype(_f32)
    onehot_cT = (_iota(P, K_FG, 0).astype(_f32) == c_row).astype(_f32)
    a_headT = _mmd(m, onehot_rT)
    a_tailT = _mmd(m, onehot_cT)
    t1 = jnp.max(relf[:, :, None] * a_headT[:, None, :], axis=0)
    lab_row = jnp.max(t1 * a_tailT, axis=0, keepdims=True)
    lab_ord = s_fg["order_row"](lab_row)

    fg_valid = (fg_val > 0).astype(_f32)
    fg_lab = lab_ord * fg_valid
    quality = fg_val * fg_valid

    valid_p = (plab_col != 0) & (plab_row != 0)
    bgmask = valid_p & (eyeP == 0.0) & (fg_mask == 0.0)
    k_bg = jnp.where(bgmask, nbits, -1)
    s_bg = _select_topk(k_bg, K_BG)
    bg_head = s_bg["order"](s_bg["r_col"])
    bg_tail = s_bg["order"](s_bg["c_col"])

    pairs_ref[0:K_FG, 0:1] = fg_head.astype(_i32)
    pairs_ref[0:K_FG, 1:2] = fg_tail.astype(_i32)
    pairs_ref[K_FG:, 0:1] = bg_head.astype(_i32)
    pairs_ref[K_FG:, 1:2] = bg_tail.astype(_i32)
    labels_ref[0:K_FG, :] = fg_lab.astype(_i32)
    labels_ref[K_FG:, :] = jnp.zeros((K_BG, 1), _i32)
    qual_ref[...] = quality


@jax.jit
def kernel(prp_boxes, prp_labels, tgt_boxes, tgt_labels, tgt_rel_matrix):
    noise = jax.random.uniform(jax.random.key(42), (P * P,))
    nbits = lax.bitcast_convert_type(noise, _i32).reshape(P, P)
    relf = tgt_rel_matrix.astype(_f32)
    args = (
        tgt_boxes.astype(_f32),
        tgt_boxes.astype(_f32).T,
        prp_boxes.astype(_f32),
        prp_boxes.astype(_f32).T,
        tgt_labels.astype(_i32).reshape(T, 1),
        tgt_labels.astype(_i32).reshape(1, T),
        prp_labels.astype(_i32).reshape(1, P),
        prp_labels.astype(_i32).reshape(P, 1),
        relf,
        relf.T,
        nbits,
    )
    out_shape = (
        jax.ShapeDtypeStruct((K_FG + K_BG, 2), _i32),
        jax.ShapeDtypeStruct((K_FG + K_BG, 1), _i32),
        jax.ShapeDtypeStruct((P, P), _i32),
        jax.ShapeDtypeStruct((K_FG, 1), _f32),
    )
    scratch = [
        pltpu.VMEM((T, P), _f32),
        pltpu.VMEM((T, P), _f32),
        pltpu.VMEM((P, P), _f32),
    ]
    pairs, labels, binary, qual = pl.pallas_call(
        _body, out_shape=out_shape, scratch_shapes=scratch)(*args)
    return pairs, labels.reshape(K_FG + K_BG), binary, qual.reshape(K_FG)


# speedup vs baseline: 8.2201x; 1.3178x over previous
import jax
import jax.numpy as jnp
from jax import lax
from jax.experimental import pallas as pl
from jax.experimental.pallas import tpu as pltpu

P = 512
T = 64
K_FG = 256
K_BG = 768
FG_THRES = 0.5

_f32 = jnp.float32
_i32 = jnp.int32


def _mmd(a, b):
    return lax.dot_general(a, b, (((1,), (0,)), ((), ())),
                           precision=lax.Precision.DEFAULT,
                           preferred_element_type=_f32)


def _rowT(col, eye_n):
    return lax.dot_general(col, eye_n, (((0,), (0,)), ((), ())),
                           precision=lax.Precision.DEFAULT,
                           preferred_element_type=_f32)


def _iota(n0, n1, d):
    return lax.broadcasted_iota(_i32, (n0, n1), d)


def _eye(n):
    return (_iota(n, n, 0) == _iota(n, n, 1)).astype(_f32)


def _split8(x):
    hi = jnp.floor(x * (1.0 / 256.0))
    return hi, x - 256.0 * hi


def _prefix_col(Lst, col):
    hi, lo = _split8(col)
    return 256.0 * _mmd(Lst, hi) + _mmd(Lst, lo)


def _select_topk(K, tau, k):
    kf = _f32(k)
    gt = (K > tau).astype(_f32)
    eq = (K == tau).astype(_f32)
    need = kf - jnp.sum(gt)

    U = (_iota(P, P, 0) < _iota(P, P, 1)).astype(_f32)
    Lst = (_iota(P, P, 0) > _iota(P, P, 1)).astype(_f32)

    w_eq = _mmd(eq, U)
    rowbase_eq = _prefix_col(Lst, jnp.sum(eq, axis=1, keepdims=True))
    sel = gt + eq * ((rowbase_eq + w_eq) < need).astype(_f32)

    w_sel = _mmd(sel, U)
    w_hi, w_lo = _split8(w_sel)
    rowcnt = jnp.sum(sel, axis=1, keepdims=True)
    rowbase = _prefix_col(Lst, rowcnt)
    re_hi, re_lo = _split8(rowbase + rowcnt)
    eyeP = _eye(P)
    rowend_row = 256.0 * _rowT(re_hi, eyeP) + _rowT(re_lo, eyeP)

    s_col = _iota(k, 1, 0).astype(_f32)
    r_col = jnp.sum((rowend_row <= s_col).astype(_f32), axis=1, keepdims=True)
    lane_k = _iota(k, P, 1).astype(_f32)
    onehot_r = (lane_k == r_col).astype(_f32)

    rb_hi, rb_lo = _split8(rowbase)
    rbase_s = 256.0 * _mmd(onehot_r, rb_hi) + _mmd(onehot_r, rb_lo)
    t_hi, t_lo = _split8(s_col - rbase_s)
    gsel = _mmd(onehot_r, sel)
    gw_hi = _mmd(onehot_r, w_hi)
    gw_lo = _mmd(onehot_r, w_lo)
    ind = gsel * ((gw_hi == t_hi) & (gw_lo == t_lo)).astype(_f32)
    c_col = jnp.sum(ind * lane_k, axis=1, keepdims=True)
    onehot_c = (lane_k == c_col).astype(_f32)

    ch = (jnp.right_shift(K, 22).astype(_f32),
          (jnp.right_shift(K, 14) & 0xFF).astype(_f32),
          (jnp.right_shift(K, 6) & 0xFF).astype(_f32),
          (K & 0x3F).astype(_f32))
    g = tuple(jnp.sum(_mmd(onehot_r, c) * ind, axis=1, keepdims=True)
              for c in ch)
    eye_k = _eye(k)
    r = tuple(_rowT(gc, eye_k) for gc in g)

    t = g[3] > r[3]
    for j in (2, 1, 0):
        t = (g[j] > r[j]) | ((g[j] == r[j]) & t)
    ceq = ((g[0] == r[0]) & (g[1] == r[1]) &
           (g[2] == r[2]) & (g[3] == r[3]))
    cmpm = t | (ceq & (_iota(k, 1, 0) < _iota(1, k, 1)))
    rank_row = jnp.sum(cmpm.astype(_f32), axis=0, keepdims=True)
    pmat = (rank_row == _iota(k, 1, 0).astype(_f32)).astype(_f32)

    return dict(pmat=pmat, chunks=g, r_col=r_col, c_col=c_col,
                onehot_r=onehot_r, onehot_c=onehot_c)


def _order_cols(pmat, cols):
    X = jnp.concatenate(cols, axis=1)
    O = _mmd(pmat, X)
    return [O[:, i:i + 1] for i in range(len(cols))]


def _body(tb_ref, tbT_ref, pb_ref, pbT_ref, tl_col_ref, tl_row_ref,
          plab_row_ref, plab_col_ref, relf_ref, relfT_ref, nbits_ref,
          pairs_ref, labels_ref, binary_ref, qual_ref,
          ious_s, inner_s, ioupair_s):
    tb = tb_ref[...]
    tbT = tbT_ref[...]
    pb = pb_ref[...]
    pbT = pbT_ref[...]
    tl_col = tl_col_ref[...]
    tl_row = tl_row_ref[...]
    plab_row = plab_row_ref[...]
    plab_col = plab_col_ref[...]
    relf = relf_ref[...]
    relfT = relfT_ref[...]
    nbits = nbits_ref[...]

    def iou(ac, bc):
        a0, a1, a2, a3 = ac
        b0, b1, b2, b3 = bc
        area_a = (a2 - a0) * (a3 - a1)
        area_b = (b2 - b0) * (b3 - b1)
        w = jnp.maximum(jnp.minimum(a2, b2) - jnp.maximum(a0, b0), 0.0)
        h = jnp.maximum(jnp.minimum(a3, b3) - jnp.maximum(a1, b1), 0.0)
        inter = w * h
        union = area_a + area_b - inter
        return inter / jnp.maximum(union, 1e-8)

    ious = iou([tb[:, i:i + 1] for i in range(4)],
               [pbT[i:i + 1, :] for i in range(4)])
    iousT = iou([pb[:, i:i + 1] for i in range(4)],
                [tbT[i:i + 1, :] for i in range(4)])

    m = ((tl_col == plab_row) & (ious > FG_THRES)).astype(_f32)
    mT = ((plab_col == tl_row) & (iousT > FG_THRES)).astype(_f32)

    rel_pos = (relf > 0).astype(_f32)
    rel_posT = (relfT > 0).astype(_f32)

    F = _mmd(_mmd(mT, rel_pos), m)
    G = _mmd(_mmd(mT, rel_posT), m)
    binary_ref[...] = ((F + G) > 0).astype(_i32)
    eyeP = _eye(P)
    fg_mask = ((F > 0).astype(_f32)) * (1.0 - eyeP)

    inner_s[...] = jnp.max(rel_posT[:, :, None] * ious[:, None, :], axis=0)
    ious_s[...] = ious
    ioupair_s[...] = jnp.zeros((P, P), _f32)

    def ip_body(i, _):
        blk = ious_s[pl.ds(i * 8, 8), :]
        ib = inner_s[pl.ds(i * 8, 8), :]
        mx = jnp.max(blk[:, :, None] * ib[:, None, :], axis=0)
        ioupair_s[...] = jnp.maximum(ioupair_s[...], mx)
        return 0

    lax.fori_loop(0, T // 8, ip_body, 0)
    iou_pair = ioupair_s[...]

    fg_scores = fg_mask * iou_pair
    k_fg = lax.bitcast_convert_type(fg_scores, _i32)

    valid_p = (plab_col != 0) & (plab_row != 0)
    bgmask = valid_p & (eyeP == 0.0) & (fg_mask == 0.0)
    k_bg = jnp.where(bgmask, nbits, -1)

    kf_fg, kf_bg = _f32(K_FG), _f32(K_BG)

    def bs_body(_, carry):
        lo_f, hi_f, lo_b, hi_b = carry
        mid_f = lo_f + lax.shift_right_logical(hi_f - lo_f + 1, 1)
        mid_b = lo_b + lax.shift_right_logical(hi_b - lo_b + 1, 1)
        cnt_f = jnp.sum((k_fg >= mid_f).astype(_f32))
        cnt_b = jnp.sum((k_bg >= mid_b).astype(_f32))
        p_f = cnt_f >= kf_fg
        p_b = cnt_b >= kf_bg
        return (jnp.where(p_f, mid_f, lo_f), jnp.where(p_f, hi_f, mid_f - 1),
                jnp.where(p_b, mid_b, lo_b), jnp.where(p_b, hi_b, mid_b - 1))

    tau_f, _, tau_b, _ = lax.fori_loop(
        0, 31, bs_body,
        (jnp.int32(-1), jnp.int32(1 << 30), jnp.int32(-1), jnp.int32(1 << 30)))

    s_fg = _select_topk(k_fg, tau_f, K_FG)
    a_head = _mmd(s_fg["onehot_r"], mT)
    a_tail = _mmd(s_fg["onehot_c"], mT)
    c1m = jnp.max(a_tail[:, None, :] * relf[None, :, :], axis=2)
    lab_col = jnp.max(c1m * a_head, axis=1, keepdims=True)

    g3, g2, g1, g0 = s_fg["chunks"]
    rh, rl = _split8(s_fg["r_col"])
    chh, cl = _split8(s_fg["c_col"])
    (o_rh, o_rl, o_ch, o_cl, o3, o2, o1, o0, o_lab) = _order_cols(
        s_fg["pmat"], [rh, rl, chh, cl, g3, g2, g1, g0, lab_col])
    fg_head = 256.0 * o_rh + o_rl
    fg_tail = 256.0 * o_ch + o_cl
    k_ord = (lax.shift_left(o3.astype(_i32), 22) |
             lax.shift_left(o2.astype(_i32), 14) |
             lax.shift_left(o1.astype(_i32), 6) | o0.astype(_i32))
    fg_val = lax.bitcast_convert_type(k_ord, _f32)
    fg_valid = (fg_val > 0).astype(_f32)
    fg_lab = o_lab * fg_valid
    quality = fg_val * fg_valid

    s_bg = _select_topk(k_bg, tau_b, K_BG)
    brh, brl = _split8(s_bg["r_col"])
    bch, bcl = _split8(s_bg["c_col"])
    o_brh, o_brl, o_bch, o_bcl = _order_cols(
        s_bg["pmat"], [brh, brl, bch, bcl])
    bg_head = 256.0 * o_brh + o_brl
    bg_tail = 256.0 * o_bch + o_bcl

    pairs_ref[0:K_FG, 0:1] = fg_head.astype(_i32)
    pairs_ref[0:K_FG, 1:2] = fg_tail.astype(_i32)
    pairs_ref[K_FG:, 0:1] = bg_head.astype(_i32)
    pairs_ref[K_FG:, 1:2] = bg_tail.astype(_i32)
    labels_ref[0:K_FG, :] = fg_lab.astype(_i32)
    labels_ref[K_FG:, :] = jnp.zeros((K_BG, 1), _i32)
    qual_ref[...] = quality


@jax.jit
def kernel(prp_boxes, prp_labels, tgt_boxes, tgt_labels, tgt_rel_matrix):
    noise = jax.random.uniform(jax.random.key(42), (P * P,))
    nbits = lax.bitcast_convert_type(noise, _i32).reshape(P, P)
    relf = tgt_rel_matrix.astype(_f32)
    args = (
        tgt_boxes.astype(_f32),
        tgt_boxes.astype(_f32).T,
        prp_boxes.astype(_f32),
        prp_boxes.astype(_f32).T,
        tgt_labels.astype(_i32).reshape(T, 1),
        tgt_labels.astype(_i32).reshape(1, T),
        prp_labels.astype(_i32).reshape(1, P),
        prp_labels.astype(_i32).reshape(P, 1),
        relf,
        relf.T,
        nbits,
    )
    out_shape = (
        jax.ShapeDtypeStruct((K_FG + K_BG, 2), _i32),
        jax.ShapeDtypeStruct((K_FG + K_BG, 1), _i32),
        jax.ShapeDtypeStruct((P, P), _i32),
        jax.ShapeDtypeStruct((K_FG, 1), _f32),
    )
    scratch = [
        pltpu.VMEM((T, P), _f32),
        pltpu.VMEM((T, P), _f32),
        pltpu.VMEM((P, P), _f32),
    ]
    pairs, labels, binary, qual = pl.pallas_call(
        _body, out_shape=out_shape, scratch_shapes=scratch)(*args)
    return pairs, labels.reshape(K_FG + K_BG), binary, qual.reshape(K_FG)
